# baseline (speedup 1.0000x reference)
import functools

import jax
import jax.numpy as jnp
from jax import lax
from jax.experimental import pallas as pl
from jax.experimental.pallas import tpu as pltpu
from jax.experimental.pallas import tpu_sc as plsc

V = 100000
D = 64
A = 128
B = 4096
NEG = 5

_info = plsc.get_sparse_core_info()
NC, NS, L = _info.num_cores, _info.num_subcores, _info.num_lanes
NW = NC * NS
NB = B // NW
C = 32
NCHUNK = NB // C


def _sc_scores(pos_u, pos_v, neg_vT, W_dense, W_aug_u, W_aug_v):
    mesh = plsc.VectorSubcoreMesh(core_axis_name="c", subcore_axis_name="s")

    @functools.partial(
        pl.kernel,
        mesh=mesh,
        compiler_params=pltpu.CompilerParams(use_tc_tiling_on_sc=False),
        out_type=[
            jax.ShapeDtypeStruct((B * L,), jnp.float32),
            jax.ShapeDtypeStruct((NEG * B * L,), jnp.float32),
        ],
        scratch_types=[
            pltpu.VMEM((NB,), jnp.int32),
            pltpu.VMEM((NB,), jnp.int32),
            [pltpu.VMEM((NB,), jnp.int32) for _ in range(NEG)],
            [[pltpu.VMEM((C, A), jnp.float32),
              pltpu.VMEM((C, A), jnp.float32),
              pltpu.VMEM((C, A), jnp.float32),
              pltpu.VMEM((C, A), jnp.float32),
              pltpu.VMEM((NEG * C, A), jnp.float32),
              pltpu.VMEM((NEG * C, A), jnp.float32),
              ] for _ in range(2)],
            pltpu.VMEM((NB * L,), jnp.float32),
            pltpu.VMEM((NEG * NB * L,), jnp.float32),
            pltpu.SemaphoreType.DMA,
            pltpu.SemaphoreType.DMA,
        ],
    )
    def k(pu_hbm, pv_hbm, nvT_hbm, wd, wau, wav,
          pos_out, neg_out,
          idx_u, idx_v, idx_n, rowbufs, pos_s, neg_s,
          sem, osem):
        wid = lax.axis_index("s") * NC + lax.axis_index("c")
        base = wid * NB

        idx_copies = [
            pltpu.async_copy(pu_hbm.at[pl.ds(base, NB)], idx_u, sem),
            pltpu.async_copy(pv_hbm.at[pl.ds(base, NB)], idx_v, sem),
        ]
        for kk in range(NEG):
            idx_copies.append(
                pltpu.async_copy(nvT_hbm.at[pl.ds(kk * B + base, NB)],
                                 idx_n[kk], sem))
        for cp in idx_copies:
            cp.wait()

        out_copies = []

        def fire(c):
            u1, u2, v1, v2, n1, n2 = rowbufs[c % 2]
            csl = pl.ds(c * C, C)
            copies = [
                pltpu.async_copy(wd.at[idx_u.at[csl]], u1, sem),
                pltpu.async_copy(wau.at[idx_u.at[csl]], u2, sem),
                pltpu.async_copy(wd.at[idx_v.at[csl]], v1, sem),
                pltpu.async_copy(wav.at[idx_v.at[csl]], v2, sem),
            ]
            for kk in range(NEG):
                copies.append(
                    pltpu.async_copy(wd.at[idx_n[kk].at[csl]],
                                     n1.at[pl.ds(kk * C, C)], sem))
                copies.append(
                    pltpu.async_copy(wav.at[idx_n[kk].at[csl]],
                                     n2.at[pl.ds(kk * C, C)], sem))
            return copies

        def compute(c):
            u1, u2, v1, v2, n1, n2 = rowbufs[c % 2]

            @plsc.parallel_loop(0, C, 1, unroll=2)
            def _(s):
                u_d = [u1[s, pl.ds(L * j, L)] for j in range(D // L)]
                u_a = [u2[s, pl.ds(L * j, L)] for j in range(A // L)]

                def dotvec(tab_d, tab_a, row):
                    acc = u_d[0] * tab_d[row, pl.ds(D, L)]
                    for j in range(1, D // L):
                        acc += u_d[j] * tab_d[row, pl.ds(D + L * j, L)]
                    for j in range(A // L):
                        acc += u_a[j] * tab_a[row, pl.ds(L * j, L)]
                    return acc

                so = c * C + s
                pos_s[pl.ds(so * L, L)] = dotvec(v1, v2, s)
                for kk in range(NEG):
                    neg_s[pl.ds((kk * NB + so) * L, L)] = dotvec(
                        n1, n2, kk * C + s)

            out_copies.append(
                pltpu.async_copy(
                    pos_s.at[pl.ds(c * C * L, C * L)],
                    pos_out.at[pl.ds((base + c * C) * L, C * L)], osem))
            for kk in range(NEG):
                out_copies.append(
                    pltpu.async_copy(
                        neg_s.at[pl.ds((kk * NB + c * C) * L, C * L)],
                        neg_out.at[pl.ds((kk * B + base + c * C) * L, C * L)],
                        osem))

        pending = fire(0)
        for c in range(NCHUNK):
            nxt = fire(c + 1) if c + 1 < NCHUNK else []
            for cp in pending:
                cp.wait()
            compute(c)
            pending = nxt
        for cp in out_copies:
            cp.wait()

    return k(pos_u, pos_v, neg_vT, W_dense, W_aug_u, W_aug_v)


_TBLK = 16384


def _build_dense(Wt_u, Wt_v):
    def body(tu_ref, tv_ref, o_ref):
        o_ref[...] = jnp.concatenate(
            [tu_ref[...].T, tv_ref[...].T], axis=-1)

    return pl.pallas_call(
        body,
        grid=(pl.cdiv(V, _TBLK),),
        in_specs=[pl.BlockSpec((D, _TBLK), lambda g: (0, g)),
                  pl.BlockSpec((D, _TBLK), lambda g: (0, g))],
        out_specs=pl.BlockSpec((_TBLK, 2 * D), lambda g: (g, 0)),
        out_shape=jax.ShapeDtypeStruct((V, 2 * D), jnp.float32),
    )(Wt_u, Wt_v)


def _final_loss(pos_part, neg_part):
    p2 = pos_part.reshape(B * L // 128, 128)
    n2 = neg_part.reshape(NEG * B * L // 128, 128)

    def body(p_ref, n_ref, o_ref):
        sel = (lax.broadcasted_iota(jnp.int32, (128, 8), 0) // L
               == lax.broadcasted_iota(jnp.int32, (128, 8), 1)
               ).astype(jnp.float32)
        p = jax.lax.dot(p_ref[...], sel,
                        preferred_element_type=jnp.float32)
        n = jax.lax.dot(n_ref[...], sel,
                        preferred_element_type=jnp.float32)
        p = jnp.clip(p, -10.0, 10.0)
        n = jnp.clip(n, -10.0, 10.0)
        pos_loss = -jax.nn.log_sigmoid(p)
        neg_loss = -jax.nn.log_sigmoid(-n)
        o_ref[0, 0] = (jnp.sum(pos_loss) + jnp.sum(neg_loss)) / B

    out = pl.pallas_call(
        body,
        out_shape=jax.ShapeDtypeStruct((1, 1), jnp.float32),
        out_specs=pl.BlockSpec(memory_space=pltpu.SMEM),
    )(p2, n2)
    return out.reshape(())


def kernel(pos_u, pos_v, neg_v, W_u1, W_v1, W_u2, W_v2):
    neg_vT = neg_v.T.reshape(-1)
    W_dense = _build_dense(W_u1.T, W_v1.T)
    pos_part, neg_part = _sc_scores(
        pos_u, pos_v, neg_vT, W_dense, W_u2, W_v2)
    return _final_loss(pos_part, neg_part)


# speedup vs baseline: 1.0386x; 1.0386x over previous
import functools

import jax
import jax.numpy as jnp
from jax import lax
from jax.experimental import pallas as pl
from jax.experimental.pallas import tpu as pltpu
from jax.experimental.pallas import tpu_sc as plsc

V = 100000
D = 64
A = 128
B = 4096
NEG = 5

_info = plsc.get_sparse_core_info()
NC, NS, L = _info.num_cores, _info.num_subcores, _info.num_lanes
NW = NC * NS
NB = B // NW
C = 32
NCHUNK = NB // C


def _sc_scores(pos_u, pos_v, neg_vT, W_dense, W_aug_u, W_aug_v):
    mesh = plsc.VectorSubcoreMesh(core_axis_name="c", subcore_axis_name="s")

    @functools.partial(
        pl.kernel,
        mesh=mesh,
        compiler_params=pltpu.CompilerParams(use_tc_tiling_on_sc=False),
        out_type=[
            jax.ShapeDtypeStruct((B * L,), jnp.float32),
            jax.ShapeDtypeStruct((NEG * B * L,), jnp.float32),
        ],
        scratch_types=[
            pltpu.VMEM((NB,), jnp.int32),
            pltpu.VMEM((NB,), jnp.int32),
            [pltpu.VMEM((NB,), jnp.int32) for _ in range(NEG)],
            pltpu.VMEM((NB,), jnp.int32),
            pltpu.VMEM((NB,), jnp.int32),
            [pltpu.VMEM((NB,), jnp.int32) for _ in range(NEG)],
            [[pltpu.VMEM((C, D), jnp.float32),
              pltpu.VMEM((C, A), jnp.float32),
              pltpu.VMEM((C, D), jnp.float32),
              pltpu.VMEM((C, A), jnp.float32),
              pltpu.VMEM((NEG * C, D), jnp.float32),
              pltpu.VMEM((NEG * C, A), jnp.float32),
              ] for _ in range(2)],
            pltpu.VMEM((NB * L,), jnp.float32),
            pltpu.VMEM((NEG * NB * L,), jnp.float32),
            pltpu.SemaphoreType.DMA,
            pltpu.SemaphoreType.DMA,
        ],
    )
    def k(pu_hbm, pv_hbm, nvT_hbm, wd, wau, wav,
          pos_out, neg_out,
          idx_u, idx_v, idx_n, du, dv, dn, rowbufs, pos_s, neg_s,
          sem, osem):
        wid = lax.axis_index("s") * NC + lax.axis_index("c")
        base = wid * NB

        idx_copies = [
            pltpu.async_copy(pu_hbm.at[pl.ds(base, NB)], idx_u, sem),
            pltpu.async_copy(pv_hbm.at[pl.ds(base, NB)], idx_v, sem),
        ]
        for kk in range(NEG):
            idx_copies.append(
                pltpu.async_copy(nvT_hbm.at[pl.ds(kk * B + base, NB)],
                                 idx_n[kk], sem))
        for cp in idx_copies:
            cp.wait()

        for j in range(NB // L):
            sl = pl.ds(j * L, L)
            du[sl] = idx_u[sl] * 2
            dv[sl] = idx_v[sl] * 2 + 1
            for kk in range(NEG):
                dn[kk][sl] = idx_n[kk][sl] * 2 + 1

        out_copies = []

        def fire(c):
            u1, u2, v1, v2, n1, n2 = rowbufs[c % 2]
            csl = pl.ds(c * C, C)
            copies = [
                pltpu.async_copy(wd.at[du.at[csl]], u1, sem),
                pltpu.async_copy(wau.at[idx_u.at[csl]], u2, sem),
                pltpu.async_copy(wd.at[dv.at[csl]], v1, sem),
                pltpu.async_copy(wav.at[idx_v.at[csl]], v2, sem),
            ]
            for kk in range(NEG):
                copies.append(
                    pltpu.async_copy(wd.at[dn[kk].at[csl]],
                                     n1.at[pl.ds(kk * C, C)], sem))
                copies.append(
                    pltpu.async_copy(wav.at[idx_n[kk].at[csl]],
                                     n2.at[pl.ds(kk * C, C)], sem))
            return copies

        def compute(c):
            u1, u2, v1, v2, n1, n2 = rowbufs[c % 2]

            @plsc.parallel_loop(0, C, 1, unroll=2)
            def _(s):
                u_d = [u1[s, pl.ds(L * j, L)] for j in range(D // L)]
                u_a = [u2[s, pl.ds(L * j, L)] for j in range(A // L)]

                def dotvec(tab_d, tab_a, row):
                    acc = u_d[0] * tab_d[row, pl.ds(0, L)]
                    for j in range(1, D // L):
                        acc += u_d[j] * tab_d[row, pl.ds(L * j, L)]
                    for j in range(A // L):
                        acc += u_a[j] * tab_a[row, pl.ds(L * j, L)]
                    return acc

                so = c * C + s
                pos_s[pl.ds(so * L, L)] = dotvec(v1, v2, s)
                for kk in range(NEG):
                    neg_s[pl.ds((kk * NB + so) * L, L)] = dotvec(
                        n1, n2, kk * C + s)

            out_copies.append(
                pltpu.async_copy(
                    pos_s.at[pl.ds(c * C * L, C * L)],
                    pos_out.at[pl.ds((base + c * C) * L, C * L)], osem))
            for kk in range(NEG):
                out_copies.append(
                    pltpu.async_copy(
                        neg_s.at[pl.ds((kk * NB + c * C) * L, C * L)],
                        neg_out.at[pl.ds((kk * B + base + c * C) * L, C * L)],
                        osem))

        pending = fire(0)
        for c in range(NCHUNK):
            nxt = fire(c + 1) if c + 1 < NCHUNK else []
            for cp in pending:
                cp.wait()
            compute(c)
            pending = nxt
        for cp in out_copies:
            cp.wait()

    return k(pos_u, pos_v, neg_vT, W_dense, W_aug_u, W_aug_v)


_TBLK = 8192


def _build_dense(Wt_u, Wt_v):
    def body(tu_ref, tv_ref, o_ref):
        o_ref[...] = jnp.concatenate(
            [tu_ref[...].T, tv_ref[...].T], axis=-1)

    return pl.pallas_call(
        body,
        grid=(pl.cdiv(V, _TBLK),),
        in_specs=[pl.BlockSpec((D, _TBLK), lambda g: (0, g)),
                  pl.BlockSpec((D, _TBLK), lambda g: (0, g))],
        out_specs=pl.BlockSpec((_TBLK, 2 * D), lambda g: (g, 0)),
        out_shape=jax.ShapeDtypeStruct((V, 2 * D), jnp.float32),
    )(Wt_u, Wt_v)


def _final_loss(pos_part, neg_part):
    p2 = pos_part.reshape(B * L // 128, 128)
    n2 = neg_part.reshape(NEG * B * L // 128, 128)

    def body(p_ref, n_ref, o_ref):
        sel = (lax.broadcasted_iota(jnp.int32, (128, 8), 0) // L
               == lax.broadcasted_iota(jnp.int32, (128, 8), 1)
               ).astype(jnp.float32)
        p = jax.lax.dot(p_ref[...], sel,
                        preferred_element_type=jnp.float32)
        n = jax.lax.dot(n_ref[...], sel,
                        preferred_element_type=jnp.float32)
        p = jnp.clip(p, -10.0, 10.0)
        n = jnp.clip(n, -10.0, 10.0)
        pos_loss = -jax.nn.log_sigmoid(p)
        neg_loss = -jax.nn.log_sigmoid(-n)
        o_ref[0, 0] = (jnp.sum(pos_loss) + jnp.sum(neg_loss)) / B

    out = pl.pallas_call(
        body,
        out_shape=jax.ShapeDtypeStruct((1, 1), jnp.float32),
        out_specs=pl.BlockSpec(memory_space=pltpu.SMEM),
    )(p2, n2)
    return out.reshape(())


def kernel(pos_u, pos_v, neg_v, W_u1, W_v1, W_u2, W_v2):
    neg_vT = neg_v.T.reshape(-1)
    W_dense = _build_dense(W_u1.T, W_v1.T)
    Wd2 = W_dense.reshape(2 * V, D)
    pos_part, neg_part = _sc_scores(
        pos_u, pos_v, neg_vT, Wd2, W_u2, W_v2)
    return _final_loss(pos_part, neg_part)
